# Initial kernel scaffold; baseline (speedup 1.0000x reference)
import jax
import jax.numpy as jnp
from jax.experimental import pallas as pl


def kernel(pred, targ):
    raise NotImplementedError("write your pallas kernel here")


# speedup vs baseline: 3.7921x; 3.7921x over previous
import functools

import jax
import jax.numpy as jnp
from jax import lax
from jax.experimental import pallas as pl
from jax.experimental.pallas import tpu as pltpu
from jax.experimental.pallas import tpu_sc as plsc

_N = 4 * 512 * 512
_LANES = 128
_ROWS = _N // _LANES
_BR = 256
_GRID = _ROWS // _BR
_NBINS = 128
_NLEV = 4
_PAD = 64
_DUMP_P = -1.0e30
_NW = 32
_BPW = _N // _NW


def _pass_a(x_ref, t_ref, stats_ref, negb_ref, z_ref):
    step = pl.program_id(0)
    x = x_ref[...]
    t = t_ref[...]
    neg = t == 0.0
    negf = jnp.where(neg, 1.0, 0.0)
    posf = 1.0 - negf
    sig = 1.0 / (1.0 + jnp.exp(-x))
    bce = jnp.maximum(x, 0.0) - x * t + jnp.log1p(jnp.exp(-jnp.abs(x)))

    @pl.when(step == 0)
    def _():
        stats_ref[...] = jnp.zeros_like(stats_ref)
        stats_ref[3:4, :] = jnp.full((1, _LANES), jnp.inf, jnp.float32)
        stats_ref[4:5, :] = jnp.full((1, _LANES), -jnp.inf, jnp.float32)

    stats_ref[0:1, :] += jnp.sum(posf, axis=0, keepdims=True)
    stats_ref[1:2, :] += jnp.sum(posf * sig, axis=0, keepdims=True)
    stats_ref[2:3, :] += jnp.sum(posf * bce, axis=0, keepdims=True)
    stats_ref[3:4, :] = jnp.minimum(
        stats_ref[3:4, :], jnp.min(jnp.where(neg, x, jnp.inf), axis=0, keepdims=True))
    stats_ref[4:5, :] = jnp.maximum(
        stats_ref[4:5, :], jnp.max(jnp.where(neg, x, -jnp.inf), axis=0, keepdims=True))
    negb_ref[...] = jnp.sum(negf, axis=0, keepdims=True).reshape(1, 1, _LANES)
    zi = lax.bitcast_convert_type(x, jnp.int32)
    z_ref[...] = (zi & jnp.int32(-2)) | t.astype(jnp.int32)


def _pass_h(x_ref, t_ref, prm_ref, hist_ref):
    step = pl.program_id(0)
    x = x_ref[...]
    t = t_ref[...]
    lo = prm_ref[0, 0]
    inv_w = prm_ref[0, 1]
    e = jnp.where(t == 0.0, jnp.floor((x - lo) * inv_w), -1.0)

    @pl.when(step == 0)
    def _():
        hist_ref[...] = jnp.zeros_like(hist_ref)

    for b in range(_NBINS):
        row = jnp.sum(jnp.where(e == float(b), 1.0, 0.0), axis=0, keepdims=True)
        hist_ref[b:b + 1, :] += row


def _pass_s(x_ref, t_ref, prm_ref, addr_ref):
    x = x_ref[...]
    t = t_ref[...]
    neg_off = prm_ref[0, 0, 0]
    lo4 = prm_ref[0, 0, 1]
    neg = t == 0.0
    negf = jnp.where(neg, 1.0, 0.0)
    sel = neg & (x >= lo4)
    ci = lax.broadcasted_iota(jnp.int32, (_LANES, _LANES), 0)
    cj = lax.broadcasted_iota(jnp.int32, (_LANES, _LANES), 1)
    triu = jnp.where(ci < cj, 1.0, 0.0)
    ri = lax.broadcasted_iota(jnp.int32, (_BR, _BR), 0)
    rj = lax.broadcasted_iota(jnp.int32, (_BR, _BR), 1)
    ltr = jnp.where(rj < ri, 1.0, 0.0)
    within_row = jnp.dot(negf, triu, preferred_element_type=jnp.float32)
    prev_rows = jnp.sum(jnp.dot(ltr, negf, preferred_element_type=jnp.float32),
                        axis=1, keepdims=True)
    rank = prev_rows + within_row
    addr = jnp.where(sel, neg_off + rank, float(_N))
    addr_ref[...] = addr.astype(jnp.int32)


def _pass_d(g_ref, out_ref):
    step = pl.program_id(0)
    g = g_ref[...]
    tg = (g & 1).astype(jnp.float32)
    pg = lax.bitcast_convert_type(g & jnp.int32(-2), jnp.float32)
    sig = 1.0 / (1.0 + jnp.exp(-pg))
    bce = jnp.maximum(pg, 0.0) - pg * tg + jnp.log1p(jnp.exp(-jnp.abs(pg)))

    @pl.when(step == 0)
    def _():
        out_ref[...] = jnp.zeros_like(out_ref)

    out_ref[0:1, :] += jnp.sum(sig, axis=0, keepdims=True)
    out_ref[1:2, :] += jnp.sum(tg, axis=0, keepdims=True)
    out_ref[2:3, :] += jnp.sum(bce, axis=0, keepdims=True)
    out_ref[3:4, :] += jnp.sum(sig * tg, axis=0, keepdims=True)


def _sc_gather(z_pad, addr):
    mesh = plsc.VectorSubcoreMesh(core_axis_name="c", subcore_axis_name="s")

    @functools.partial(
        pl.kernel,
        out_type=jax.ShapeDtypeStruct((_N,), jnp.int32),
        mesh=mesh,
        scratch_types=[
            pltpu.VMEM((_BPW,), jnp.int32),
            pltpu.VMEM((_BPW,), jnp.int32),
            pltpu.SemaphoreType.DMA,
        ],
    )
    def _k(z_hbm, a_hbm, out_hbm, idx_v, g_v, sem):
        wid = lax.axis_index("s") * 2 + lax.axis_index("c")
        base = wid * _BPW
        pltpu.sync_copy(a_hbm.at[pl.ds(base, _BPW)], idx_v)
        pltpu.async_copy(z_hbm.at[idx_v], g_v, sem).wait()
        pltpu.sync_copy(g_v, out_hbm.at[pl.ds(base, _BPW)])

    return _k(z_pad, addr)


def _row_spec(block_rows):
    return pl.BlockSpec((block_rows, _LANES), lambda b: (b, 0))


def kernel(pred, targ):
    f32 = jnp.float32
    x = pred.reshape(_ROWS, _LANES)
    t = targ.reshape(_ROWS, _LANES)

    stats, negb, z = pl.pallas_call(
        _pass_a,
        grid=(_GRID,),
        in_specs=[_row_spec(_BR), _row_spec(_BR)],
        out_specs=[
            pl.BlockSpec((8, _LANES), lambda b: (0, 0)),
            pl.BlockSpec((1, 1, _LANES), lambda b: (b, 0, 0)),
            _row_spec(_BR),
        ],
        out_shape=[
            jax.ShapeDtypeStruct((8, _LANES), f32),
            jax.ShapeDtypeStruct((_GRID, 1, _LANES), f32),
            jax.ShapeDtypeStruct((_ROWS, _LANES), jnp.int32),
        ],
    )(x, t)

    n_pos = jnp.sum(stats[0])
    sumsig_pos = jnp.sum(stats[1])
    sumbce_pos = jnp.sum(stats[2])
    lo = jnp.min(stats[3])
    hi = jnp.max(stats[4])
    negcnt_b = jnp.sum(negb[:, 0, :], axis=1)
    neg_off_b = jnp.cumsum(negcnt_b) - negcnt_b
    k = 3.0 * n_pos

    width = (hi - lo) * (1.000001 / _NBINS) + 1e-30
    kk = k
    bins = jnp.arange(_NBINS)
    for _ in range(_NLEV):
        prm = jnp.zeros((8, _LANES), f32)
        prm = prm.at[0, 0].set(lo).at[0, 1].set(1.0 / width)
        hist2 = pl.pallas_call(
            _pass_h,
            grid=(_GRID,),
            in_specs=[_row_spec(_BR), _row_spec(_BR),
                      pl.BlockSpec((8, _LANES), lambda b: (0, 0))],
            out_specs=pl.BlockSpec((_NBINS, _LANES), lambda b: (0, 0)),
            out_shape=jax.ShapeDtypeStruct((_NBINS, _LANES), f32),
        )(x, t, prm)
        histv = jnp.sum(hist2, axis=1)
        rc = jnp.cumsum(histv[::-1])[::-1]
        bsel = jnp.max(jnp.where(rc >= kk, bins, 0))
        kk = kk - (rc[bsel] - histv[bsel])
        lo = lo + bsel.astype(f32) * width
        width = width * (1.0 / _NBINS)

    lo4 = jnp.where(k > 0, lo, jnp.inf).astype(f32)

    prm_s = jnp.zeros((_GRID, 1, _LANES), f32)
    prm_s = prm_s.at[:, 0, 0].set(neg_off_b).at[:, 0, 1].set(lo4)
    addr = pl.pallas_call(
        _pass_s,
        grid=(_GRID,),
        in_specs=[_row_spec(_BR), _row_spec(_BR),
                  pl.BlockSpec((1, 1, _LANES), lambda b: (b, 0, 0))],
        out_specs=_row_spec(_BR),
        out_shape=jax.ShapeDtypeStruct((_ROWS, _LANES), jnp.int32),
    )(x, t, prm_s)

    dump_z = lax.bitcast_convert_type(jnp.float32(_DUMP_P), jnp.int32) & jnp.int32(-2)
    z_pad = jnp.concatenate([z.reshape(_N), jnp.full((_PAD,), dump_z, jnp.int32)])
    g = _sc_gather(z_pad, addr.reshape(_N))

    sums = pl.pallas_call(
        _pass_d,
        grid=(_GRID,),
        in_specs=[_row_spec(_BR)],
        out_specs=pl.BlockSpec((8, _LANES), lambda b: (0, 0)),
        out_shape=jax.ShapeDtypeStruct((8, _LANES), f32),
    )(g.reshape(_ROWS, _LANES))

    s_sig = jnp.sum(sums[0])
    s_t = jnp.sum(sums[1])
    s_bce = jnp.sum(sums[2])
    s_sigt = jnp.sum(sums[3])

    total = 4.0 * n_pos
    inter = s_sigt + sumsig_pos
    dice = (2.0 * inter + 1.0) / ((s_sig + sumsig_pos) + (s_t + n_pos) + 1.0)
    return (1.0 - dice) + (s_bce + sumbce_pos) / total
